# baseline (speedup 1.0000x reference)
import functools

import jax
import jax.numpy as jnp
from jax import lax
from jax.experimental import pallas as pl
from jax.experimental.pallas import tpu as pltpu

_N = 50000
_F = 512
_G = 256
_STEPS = 6
_BLK = 5000
_NBLK = _N // _BLK
_W = 256
_NW = _G // _W
_MAXP = _NBLK + _NW - 1
_NEG = -1e30


def _kernel(cidx_ref, widx_ref, tot_ref,
            x_ref, ids_ref, wih_ref, whh_ref, bih_ref, bhh_ref,
            out_ref,
            h_s, c_s, q_s, m_s, d_s, r_s):
    s = pl.program_id(0)
    k = pl.program_id(1)

    @pl.when(jnp.logical_and(s == 0, k == 0))
    def _init():
        q_s[...] = jnp.zeros((_G, 2 * _F), jnp.float32)
        h_s[...] = jnp.zeros((_G, _F), jnp.float32)
        c_s[...] = jnp.zeros((_G, _F), jnp.float32)

    @pl.when(k == 0)
    def _lstm():
        gates = (
            lax.dot_general(q_s[...], wih_ref[...],
                            (((1,), (1,)), ((), ())),
                            preferred_element_type=jnp.float32)
            + lax.dot_general(h_s[...], whh_ref[...],
                              (((1,), (1,)), ((), ())),
                              preferred_element_type=jnp.float32)
            + bih_ref[...] + bhh_ref[...]
        )
        i_g = gates[:, :_F]
        f_g = gates[:, _F:2 * _F]
        g_g = gates[:, 2 * _F:3 * _F]
        o_g = gates[:, 3 * _F:]
        c_new = jax.nn.sigmoid(f_g) * c_s[...] + jax.nn.sigmoid(i_g) * jnp.tanh(g_g)
        h_new = jax.nn.sigmoid(o_g) * jnp.tanh(c_new)
        h_s[...] = h_new
        c_s[...] = c_new
        m_s[...] = jnp.full((_G, 1), _NEG, jnp.float32)
        d_s[...] = jnp.zeros((_G, 1), jnp.float32)
        r_s[...] = jnp.zeros((_G, _F), jnp.float32)

    w = widx_ref[k]
    base = w * _W

    @pl.when(k < tot_ref[0])
    def _pair():
        xb = x_ref[...]
        ids = ids_ref[0]
        hw = h_s[pl.ds(base, _W), :]
        et = lax.dot_general(hw, xb, (((1,), (1,)), ((), ())),
                             preferred_element_type=jnp.float32)
        lg = ids - base
        og = lax.broadcasted_iota(jnp.int32, (_W, _BLK), 0) == lg
        m_blk = jnp.max(jnp.where(og, et, _NEG), axis=1, keepdims=True)
        m_old = m_s[pl.ds(base, _W), :]
        m_new = jnp.maximum(m_old, m_blk)
        alpha = jnp.exp(m_old - m_new)
        p = jnp.where(og, jnp.exp(et - m_new), 0.0)
        d_blk = jnp.sum(p, axis=1, keepdims=True)
        r_s[pl.ds(base, _W), :] = r_s[pl.ds(base, _W), :] * alpha + lax.dot_general(
            p, xb, (((1,), (0,)), ((), ())), preferred_element_type=jnp.float32)
        d_s[pl.ds(base, _W), :] = d_s[pl.ds(base, _W), :] * alpha + d_blk
        m_s[pl.ds(base, _W), :] = m_new

    @pl.when(k == _MAXP - 1)
    def _finish():
        r = r_s[...] / (d_s[...] + 1e-16)
        q_s[:, :_F] = h_s[...]
        q_s[:, _F:] = r

    @pl.when(jnp.logical_and(s == _STEPS - 1, k == _MAXP - 1))
    def _emit():
        out_ref[...] = q_s[...]


@jax.jit
def _run(x, ids3, cidx, widx, tot, w_ih, w_hh, b_ih2, b_hh2):
    grid_spec = pltpu.PrefetchScalarGridSpec(
        num_scalar_prefetch=3,
        grid=(_STEPS, _MAXP),
        in_specs=[
            pl.BlockSpec((_BLK, _F), lambda s, k, ci, wi, tt: (ci[k], 0)),
            pl.BlockSpec((1, 1, _BLK), lambda s, k, ci, wi, tt: (ci[k], 0, 0)),
            pl.BlockSpec((4 * _F, 2 * _F), lambda s, k, ci, wi, tt: (0, 0)),
            pl.BlockSpec((4 * _F, _F), lambda s, k, ci, wi, tt: (0, 0)),
            pl.BlockSpec((1, 4 * _F), lambda s, k, ci, wi, tt: (0, 0)),
            pl.BlockSpec((1, 4 * _F), lambda s, k, ci, wi, tt: (0, 0)),
        ],
        out_specs=pl.BlockSpec((_G, 2 * _F), lambda s, k, ci, wi, tt: (0, 0)),
        scratch_shapes=[
            pltpu.VMEM((_G, _F), jnp.float32),
            pltpu.VMEM((_G, _F), jnp.float32),
            pltpu.VMEM((_G, 2 * _F), jnp.float32),
            pltpu.VMEM((_G, 1), jnp.float32),
            pltpu.VMEM((_G, 1), jnp.float32),
            pltpu.VMEM((_G, _F), jnp.float32),
        ],
    )
    return pl.pallas_call(
        _kernel,
        grid_spec=grid_spec,
        out_shape=jax.ShapeDtypeStruct((_G, 2 * _F), jnp.float32),
    )(cidx, widx, tot, x, ids3, w_ih, w_hh, b_ih2, b_hh2)


def _schedule(batch):
    firsts = batch[:: _BLK]
    lasts = batch[_BLK - 1:: _BLK]
    w_lo = firsts // _W
    w_hi = lasts // _W
    cnt = w_hi - w_lo + 1
    s_off = jnp.cumsum(cnt) - cnt
    total = s_off[-1] + cnt[-1]
    k = jnp.arange(_MAXP)
    cidx = jnp.clip(jnp.searchsorted(s_off, k, side="right") - 1, 0, _NBLK - 1)
    widx = jnp.clip(w_lo[cidx] + (k - s_off[cidx]), 0, _NW - 1)
    return (cidx.astype(jnp.int32), widx.astype(jnp.int32),
            total.astype(jnp.int32).reshape(1))


def kernel(x, batch, W_ih, W_hh, b_ih, b_hh):
    batch = batch.astype(jnp.int32)
    cidx, widx, tot = _schedule(batch)
    ids3 = batch.reshape(_NBLK, 1, _BLK)
    return _run(x, ids3, cidx, widx, tot, W_ih, W_hh,
                b_ih.reshape(1, -1), b_hh.reshape(1, -1))


# speedup vs baseline: 1.0725x; 1.0725x over previous
import functools

import jax
import jax.numpy as jnp
from jax import lax
from jax.experimental import pallas as pl
from jax.experimental.pallas import tpu as pltpu

_N = 50000
_F = 512
_G = 256
_STEPS = 6
_BLK = 5000
_NBLK = _N // _BLK
_W = 128
_NW = _G // _W
_MAXP = _NBLK + _NW - 1
_NEG = -1e30


def _kernel(cidx_ref, widx_ref, tot_ref,
            x_ref, ids_ref, wih_ref, whh_ref, bih_ref, bhh_ref,
            out_ref,
            h_s, c_s, q_s, m_s, d_s, r_s):
    s = pl.program_id(0)
    k = pl.program_id(1)

    @pl.when(jnp.logical_and(s == 0, k == 0))
    def _init():
        q_s[...] = jnp.zeros((_G, 2 * _F), jnp.float32)
        h_s[...] = jnp.zeros((_G, _F), jnp.float32)
        c_s[...] = jnp.zeros((_G, _F), jnp.float32)

    @pl.when(k == 0)
    def _lstm():
        gates = (
            lax.dot_general(q_s[...], wih_ref[...],
                            (((1,), (1,)), ((), ())),
                            preferred_element_type=jnp.float32)
            + lax.dot_general(h_s[...], whh_ref[...],
                              (((1,), (1,)), ((), ())),
                              preferred_element_type=jnp.float32)
            + bih_ref[...] + bhh_ref[...]
        )
        i_g = gates[:, :_F]
        f_g = gates[:, _F:2 * _F]
        g_g = gates[:, 2 * _F:3 * _F]
        o_g = gates[:, 3 * _F:]
        c_new = jax.nn.sigmoid(f_g) * c_s[...] + jax.nn.sigmoid(i_g) * jnp.tanh(g_g)
        h_new = jax.nn.sigmoid(o_g) * jnp.tanh(c_new)
        h_s[...] = h_new
        c_s[...] = c_new
        m_s[...] = jnp.full((_G, 1), _NEG, jnp.float32)
        d_s[...] = jnp.zeros((_G, 1), jnp.float32)
        r_s[...] = jnp.zeros((_G, _F), jnp.float32)

    w = widx_ref[k]
    base = w * _W

    @pl.when(k < tot_ref[0])
    def _pair():
        xb = x_ref[...]
        ids = ids_ref[0]
        hw = h_s[pl.ds(base, _W), :]
        et = lax.dot_general(hw, xb, (((1,), (1,)), ((), ())),
                             preferred_element_type=jnp.float32)
        lg = ids - base
        og = lax.broadcasted_iota(jnp.int32, (_W, _BLK), 0) == lg
        m_blk = jnp.max(jnp.where(og, et, _NEG), axis=1, keepdims=True)
        m_old = m_s[pl.ds(base, _W), :]
        m_new = jnp.maximum(m_old, m_blk)
        alpha = jnp.exp(m_old - m_new)
        p = jnp.where(og, jnp.exp(et - m_new), 0.0)
        d_blk = jnp.sum(p, axis=1, keepdims=True)
        r_s[pl.ds(base, _W), :] = r_s[pl.ds(base, _W), :] * alpha + lax.dot_general(
            p.astype(jnp.bfloat16), xb.astype(jnp.bfloat16),
            (((1,), (0,)), ((), ())), preferred_element_type=jnp.float32)
        d_s[pl.ds(base, _W), :] = d_s[pl.ds(base, _W), :] * alpha + d_blk
        m_s[pl.ds(base, _W), :] = m_new

    @pl.when(k == _MAXP - 1)
    def _finish():
        r = r_s[...] / (d_s[...] + 1e-16)
        q_s[:, :_F] = h_s[...]
        q_s[:, _F:] = r

    @pl.when(jnp.logical_and(s == _STEPS - 1, k == _MAXP - 1))
    def _emit():
        out_ref[...] = q_s[...]


@jax.jit
def _run(x, ids3, cidx, widx, tot, w_ih, w_hh, b_ih2, b_hh2):
    grid_spec = pltpu.PrefetchScalarGridSpec(
        num_scalar_prefetch=3,
        grid=(_STEPS, _MAXP),
        in_specs=[
            pl.BlockSpec((_BLK, _F), lambda s, k, ci, wi, tt: (ci[k], 0)),
            pl.BlockSpec((1, 1, _BLK), lambda s, k, ci, wi, tt: (ci[k], 0, 0)),
            pl.BlockSpec((4 * _F, 2 * _F), lambda s, k, ci, wi, tt: (0, 0)),
            pl.BlockSpec((4 * _F, _F), lambda s, k, ci, wi, tt: (0, 0)),
            pl.BlockSpec((1, 4 * _F), lambda s, k, ci, wi, tt: (0, 0)),
            pl.BlockSpec((1, 4 * _F), lambda s, k, ci, wi, tt: (0, 0)),
        ],
        out_specs=pl.BlockSpec((_G, 2 * _F), lambda s, k, ci, wi, tt: (0, 0)),
        scratch_shapes=[
            pltpu.VMEM((_G, _F), jnp.float32),
            pltpu.VMEM((_G, _F), jnp.float32),
            pltpu.VMEM((_G, 2 * _F), jnp.float32),
            pltpu.VMEM((_G, 1), jnp.float32),
            pltpu.VMEM((_G, 1), jnp.float32),
            pltpu.VMEM((_G, _F), jnp.float32),
        ],
    )
    return pl.pallas_call(
        _kernel,
        grid_spec=grid_spec,
        out_shape=jax.ShapeDtypeStruct((_G, 2 * _F), jnp.float32),
    )(cidx, widx, tot, x, ids3, w_ih, w_hh, b_ih2, b_hh2)


def _schedule(batch):
    firsts = batch[:: _BLK]
    lasts = batch[_BLK - 1:: _BLK]
    w_lo = firsts // _W
    w_hi = lasts // _W
    cnt = w_hi - w_lo + 1
    s_off = jnp.cumsum(cnt) - cnt
    total = s_off[-1] + cnt[-1]
    k = jnp.arange(_MAXP)
    cidx = jnp.clip(jnp.searchsorted(s_off, k, side="right") - 1, 0, _NBLK - 1)
    widx = jnp.clip(w_lo[cidx] + (k - s_off[cidx]), 0, _NW - 1)
    return (cidx.astype(jnp.int32), widx.astype(jnp.int32),
            total.astype(jnp.int32).reshape(1))


def kernel(x, batch, W_ih, W_hh, b_ih, b_hh):
    batch = batch.astype(jnp.int32)
    cidx, widx, tot = _schedule(batch)
    ids3 = batch.reshape(_NBLK, 1, _BLK)
    return _run(x, ids3, cidx, widx, tot, W_ih, W_hh,
                b_ih.reshape(1, -1), b_hh.reshape(1, -1))
